# Initial kernel scaffold; baseline (speedup 1.0000x reference)
import jax
import jax.numpy as jnp
from jax.experimental import pallas as pl


def kernel(x, edge_index, graph_indices, W1l, W1r, b1, W2l, W2r, b2, Wpl, Wpr, bp, Wv, bv):
    raise NotImplementedError("write your pallas kernel here")


# speedup vs baseline: 8.0322x; 8.0322x over previous
import functools
import jax
import jax.numpy as jnp
from jax import lax
from jax.experimental import pallas as pl
from jax.experimental.pallas import tpu as pltpu
from jax.experimental.pallas import tpu_sc as plsc

N_NODES = 10000
N_EDGES = 320000
DIM = 128
N_GRAPHS = 16

NC = 2
NS = 16
NP = 10240
EW = N_EDGES // (NC * NS)
CH = 80
NCHUNK = EW // CH
RPT = NP // NS

RB = 1024
GRID = NP // RB
SUB = RB // DIM

_mesh = plsc.VectorSubcoreMesh(
    core_axis_name="c", subcore_axis_name="s", num_cores=NC, num_subcores=NS)


def _zero16():
    return jnp.zeros((16,), jnp.float32)


def _sc_edge_body(with_deg, *refs):
    if with_deg:
        (p_hbm, src_hbm, dst_hbm, acc_out, deg_out,
         srcv, dstv, rows, ones_v, acc_sh, deg_sh) = refs
    else:
        (p_hbm, src_hbm, dst_hbm, acc_out,
         srcv, dstv, rows, ones_v, acc_sh, deg_sh) = refs
        deg_out = None

    cid = lax.axis_index("c")
    sid = lax.axis_index("s")
    base = pl.multiple_of(sid * RPT, RPT)

    pltpu.sync_copy(src_hbm.at[cid, sid], srcv)
    pltpu.sync_copy(dst_hbm.at[cid, sid], dstv)

    def _zrow(i, c):
        for k in range(DIM // 16):
            rows[i, pl.ds(k * 16, 16)] = _zero16()
        return c
    lax.fori_loop(0, CH, _zrow, 0)
    for k in range(CH // 16):
        ones_v[pl.ds(k * 16, 16)] = jnp.ones((16,), jnp.float32)
    for t in range(RPT // CH):
        pltpu.sync_copy(rows, acc_sh.at[pl.ds(base + t * CH, CH)])
    if with_deg:
        zvec = rows.at[0]
        for t in range(RPT // DIM):
            pltpu.sync_copy(zvec, deg_sh.at[pl.ds(base + t * DIM, DIM)])
    plsc.subcore_barrier()

    def _chunk(j, c):
        pltpu.sync_copy(p_hbm.at[srcv.at[j]], rows)
        pltpu.sync_copy(rows, acc_sh.at[dstv.at[j]], add=True)
        if with_deg:
            pltpu.sync_copy(ones_v, deg_sh.at[dstv.at[j]], add=True)
        return c
    lax.fori_loop(0, NCHUNK, _chunk, 0)
    plsc.subcore_barrier()

    pltpu.sync_copy(acc_sh.at[pl.ds(base, RPT)],
                    acc_out.at[cid, pl.ds(base, RPT)])
    if with_deg:
        pltpu.sync_copy(deg_sh.at[pl.ds(base, RPT)],
                        deg_out.at[cid, pl.ds(base, RPT)])


def _make_sc_edge(with_deg):
    out_type = [jax.ShapeDtypeStruct((NC, NP, DIM), jnp.float32)]
    if with_deg:
        out_type.append(jax.ShapeDtypeStruct((NC, NP), jnp.float32))
    return pl.kernel(
        functools.partial(_sc_edge_body, with_deg),
        out_type=out_type,
        mesh=_mesh,
        scratch_types=[
            pltpu.VMEM((NCHUNK, CH), jnp.int32),
            pltpu.VMEM((NCHUNK, CH), jnp.int32),
            pltpu.VMEM((CH, DIM), jnp.float32),
            pltpu.VMEM((CH,), jnp.float32),
            pltpu.VMEM_SHARED((NP, DIM), jnp.float32),
            pltpu.VMEM_SHARED((NP,), jnp.float32),
        ],
        name="sc_edge_segsum" + ("_deg" if with_deg else ""),
    )


_sc_edge_deg = _make_sc_edge(True)
_sc_edge = _make_sc_edge(False)


def _sc_scalar_body(q_hbm, src_hbm, dst_hbm, accq_out,
                    srcv, dstv, qrows, dacc):
    cid = lax.axis_index("c")
    sid = lax.axis_index("s")
    base = pl.multiple_of(sid * RPT, RPT)

    pltpu.sync_copy(src_hbm.at[cid, sid], srcv)
    pltpu.sync_copy(dst_hbm.at[cid, sid], dstv)

    for k in range(CH // 16):
        qrows[pl.ds(k * 16, 16)] = _zero16()
    for t in range(RPT // CH):
        pltpu.sync_copy(qrows, dacc.at[pl.ds(base + t * CH, CH)])
    plsc.subcore_barrier()

    def _chunk(j, c):
        pltpu.sync_copy(q_hbm.at[srcv.at[j]], qrows)
        pltpu.sync_copy(qrows, dacc.at[dstv.at[j]], add=True)
        return c
    lax.fori_loop(0, NCHUNK, _chunk, 0)
    plsc.subcore_barrier()

    pltpu.sync_copy(dacc.at[pl.ds(base, RPT)],
                    accq_out.at[cid, pl.ds(base, RPT)])


_sc_scalar = pl.kernel(
    _sc_scalar_body,
    out_type=jax.ShapeDtypeStruct((NC, NP), jnp.float32),
    mesh=_mesh,
    scratch_types=[
        pltpu.VMEM((NCHUNK, CH), jnp.int32),
        pltpu.VMEM((NCHUNK, CH), jnp.int32),
        pltpu.VMEM((CH,), jnp.float32),
        pltpu.VMEM_SHARED((NP,), jnp.float32),
    ],
    name="sc_scalar_segsum",
)


def _mm_body(x_ref, w_ref, o_ref):
    o_ref[...] = jnp.dot(x_ref[...], w_ref[...],
                         preferred_element_type=jnp.float32)


_mm = pl.pallas_call(
    _mm_body,
    grid=(GRID,),
    in_specs=[
        pl.BlockSpec((RB, DIM), lambda i: (i, 0)),
        pl.BlockSpec((DIM, DIM), lambda i: (0, 0)),
    ],
    out_specs=pl.BlockSpec((RB, DIM), lambda i: (i, 0)),
    out_shape=jax.ShapeDtypeStruct((NP, DIM), jnp.float32),
)


def _eye():
    return (lax.broadcasted_iota(jnp.int32, (DIM, DIM), 0)
            == lax.broadcasted_iota(jnp.int32, (DIM, DIM), 1)
            ).astype(jnp.float32)


def _cols_of(rows):
    return lax.dot_general(_eye(), rows, (((1,), (1,)), ((), ())),
                           preferred_element_type=jnp.float32)


def _rows_of(cols):
    return lax.dot_general(cols, _eye(), (((0,), (0,)), ((), ())),
                           preferred_element_type=jnp.float32)


def _tcb_body(acc_ref, degp_ref, x_ref, w1r_ref, w2l_ref, b1_ref,
              h1_ref, p2_ref, invd_ref):
    deg = jnp.maximum(degp_ref[0] + degp_ref[1], 1.0)
    inv = 1.0 / deg
    invd_ref[...] = inv
    invT = _cols_of(inv)
    accs = acc_ref[0] + acc_ref[1]
    xr = (jnp.dot(x_ref[...], w1r_ref[...],
                  preferred_element_type=jnp.float32) + b1_ref[...])
    for s in range(SUB):
        mean_s = accs[s * DIM:(s + 1) * DIM, :] * invT[:, s:s + 1]
        h1_ref[pl.ds(s * DIM, DIM), :] = jnp.maximum(
            mean_s + xr[s * DIM:(s + 1) * DIM, :], 0.0)
    p2_ref[...] = jnp.dot(h1_ref[...], w2l_ref[...],
                          preferred_element_type=jnp.float32)


_tcb = pl.pallas_call(
    _tcb_body,
    grid=(GRID,),
    in_specs=[
        pl.BlockSpec((NC, RB, DIM), lambda i: (0, i, 0)),
        pl.BlockSpec((NC, SUB, DIM), lambda i: (0, i, 0)),
        pl.BlockSpec((RB, DIM), lambda i: (i, 0)),
        pl.BlockSpec((DIM, DIM), lambda i: (0, 0)),
        pl.BlockSpec((DIM, DIM), lambda i: (0, 0)),
        pl.BlockSpec((1, DIM), lambda i: (0, 0)),
    ],
    out_specs=[
        pl.BlockSpec((RB, DIM), lambda i: (i, 0)),
        pl.BlockSpec((RB, DIM), lambda i: (i, 0)),
        pl.BlockSpec((SUB, DIM), lambda i: (i, 0)),
    ],
    out_shape=[
        jax.ShapeDtypeStruct((NP, DIM), jnp.float32),
        jax.ShapeDtypeStruct((NP, DIM), jnp.float32),
        jax.ShapeDtypeStruct((NP // DIM, DIM), jnp.float32),
    ],
)


def _tcc_body(acc_ref, invd_ref, h1_ref, w2r_ref, b2_ref, wp_ref, gi_ref,
              q_ref, rp_ref, gp_ref):
    i = pl.program_id(0)
    invT = _cols_of(invd_ref[...])
    giT = _cols_of(gi_ref[...].astype(jnp.float32))
    accs = acc_ref[0] + acc_ref[1]
    hr = (jnp.dot(h1_ref[...], w2r_ref[...],
                  preferred_element_type=jnp.float32) + b2_ref[...])
    io = lax.broadcasted_iota(jnp.int32, (DIM, N_GRAPHS), 1).astype(jnp.float32)
    h2_parts = []
    oh_parts = []
    for s in range(SUB):
        h2_s = (accs[s * DIM:(s + 1) * DIM, :] * invT[:, s:s + 1]
                + hr[s * DIM:(s + 1) * DIM, :])
        h2_parts.append(h2_s)
        oh_parts.append((giT[:, s:s + 1] == io).astype(jnp.float32))
    h2 = jnp.concatenate(h2_parts, axis=0)
    onehot = jnp.concatenate(oh_parts, axis=0)
    qrp = jnp.dot(h2, wp_ref[...], preferred_element_type=jnp.float32)
    q_cols = jnp.concatenate(
        [qrp[s * DIM:(s + 1) * DIM, 0:1] for s in range(SUB)], axis=1)
    r_cols = jnp.concatenate(
        [qrp[s * DIM:(s + 1) * DIM, 1:2] for s in range(SUB)], axis=1)
    q_ref[...] = _rows_of(q_cols)
    rp_ref[...] = _rows_of(r_cols)
    part = lax.dot_general(onehot, h2, (((0,), (0,)), ((), ())),
                           preferred_element_type=jnp.float32)

    @pl.when(i == 0)
    def _():
        gp_ref[...] = part

    @pl.when(i > 0)
    def _():
        gp_ref[...] += part


_tcc = pl.pallas_call(
    _tcc_body,
    grid=(GRID,),
    in_specs=[
        pl.BlockSpec((NC, RB, DIM), lambda i: (0, i, 0)),
        pl.BlockSpec((SUB, DIM), lambda i: (i, 0)),
        pl.BlockSpec((RB, DIM), lambda i: (i, 0)),
        pl.BlockSpec((DIM, DIM), lambda i: (0, 0)),
        pl.BlockSpec((1, DIM), lambda i: (0, 0)),
        pl.BlockSpec((DIM, 2), lambda i: (0, 0)),
        pl.BlockSpec((SUB, DIM), lambda i: (i, 0)),
    ],
    out_specs=[
        pl.BlockSpec((SUB, DIM), lambda i: (i, 0)),
        pl.BlockSpec((SUB, DIM), lambda i: (i, 0)),
        pl.BlockSpec((N_GRAPHS, DIM), lambda i: (0, 0)),
    ],
    out_shape=[
        jax.ShapeDtypeStruct((NP // DIM, DIM), jnp.float32),
        jax.ShapeDtypeStruct((NP // DIM, DIM), jnp.float32),
        jax.ShapeDtypeStruct((N_GRAPHS, DIM), jnp.float32),
    ],
)


def _tcd_body(accq_ref, invd_ref, rp_ref, bp_ref, gp_ref, wv_ref, bv_ref,
              pol_ref, val_ref):
    accq = accq_ref[0] + accq_ref[1]
    pol_ref[...] = accq * invd_ref[...] + rp_ref[...] + bp_ref[...]
    v = jnp.sum(gp_ref[...] * wv_ref[...], axis=1, keepdims=True) + bv_ref[...]
    val_ref[...] = jnp.broadcast_to(jax.nn.sigmoid(v), (N_GRAPHS, DIM))


_tcd = pl.pallas_call(
    _tcd_body,
    grid=(1,),
    in_specs=[
        pl.BlockSpec((NC, NP // DIM, DIM), lambda i: (0, 0, 0)),
        pl.BlockSpec((NP // DIM, DIM), lambda i: (0, 0)),
        pl.BlockSpec((NP // DIM, DIM), lambda i: (0, 0)),
        pl.BlockSpec((1, 1), lambda i: (0, 0)),
        pl.BlockSpec((N_GRAPHS, DIM), lambda i: (0, 0)),
        pl.BlockSpec((1, DIM), lambda i: (0, 0)),
        pl.BlockSpec((1, 1), lambda i: (0, 0)),
    ],
    out_specs=[
        pl.BlockSpec((NP // DIM, DIM), lambda i: (0, 0)),
        pl.BlockSpec((N_GRAPHS, DIM), lambda i: (0, 0)),
    ],
    out_shape=[
        jax.ShapeDtypeStruct((NP // DIM, DIM), jnp.float32),
        jax.ShapeDtypeStruct((N_GRAPHS, DIM), jnp.float32),
    ],
)


def kernel(x, edge_index, graph_indices,
           W1l, W1r, b1, W2l, W2r, b2, Wpl, Wpr, bp, Wv, bv):
    xp = jnp.pad(x, ((0, NP - N_NODES), (0, 0)))
    src_r = edge_index[0].reshape(NC, NS, NCHUNK, CH)
    dst_r = edge_index[1].reshape(NC, NS, NCHUNK, CH)
    gi_pad = jnp.pad(graph_indices, (0, NP - N_NODES),
                     constant_values=N_GRAPHS).reshape(NP // DIM, DIM)
    b1r = b1.reshape(1, DIM)
    b2r = b2.reshape(1, DIM)
    wp = jnp.concatenate([Wpl, Wpr], axis=1)
    wv_row = Wv.reshape(1, DIM)
    bp_r = bp.reshape(1, 1)
    bv_r = bv.reshape(1, 1)

    p1 = _mm(xp, W1l)
    acc1, degp = _sc_edge_deg(p1, src_r, dst_r)
    degp_r = degp.reshape(NC, NP // DIM, DIM)
    h1, p2, invd = _tcb(acc1, degp_r, xp, W1r, W2l, b1r)
    (acc2,) = _sc_edge(p2, src_r, dst_r)
    q, rp, gp = _tcc(acc2, invd, h1, W2r, b2r, wp, gi_pad)
    accq = _sc_scalar(q.reshape(NP), src_r, dst_r)
    accq_r = accq.reshape(NC, NP // DIM, DIM)
    pol_r, val_b = _tcd(accq_r, invd, rp, bp_r, gp, wv_row, bv_r)
    policy = pol_r.reshape(NP, 1)[:N_NODES]
    value = val_b[:, 0:1]
    return (policy, value)
